# baseline (speedup 1.0000x reference)
import functools

import jax
import jax.numpy as jnp
import numpy as np
from jax import lax
from jax.experimental import pallas as pl
from jax.experimental.pallas import tpu as pltpu
from jax.experimental.pallas import tpu_sc as plsc

_VOCAB = 1000000
_EMB = 128
_B = 4096
_NDENSE = 13
_NSPARSE = 26
_BOT = 256

_NC, _NS = 2, 16
_NW = _NC * _NS
_TOT = _B * _NSPARSE
_PERW = _TOT // _NW
_CHUNK = 128
_NCHUNK = _PERW // _CHUNK
_BCHUNKS = _B // _CHUNK

def _twe2(tw0e):
  t = tw0e.reshape(_NSPARSE // 2, 2, 4, 2, 16, _BOT)
  return t.transpose(0, 3, 1, 2, 4, 5).reshape(_NSPARSE // 2, 2 * _EMB, _BOT)


def _sc_gather(table, xst, boff, nb, rows, ncores=_NC):
  chunk = 2 * rows
  btiles = nb // rows
  ntiles = (_NSPARSE // 2) * btiles
  nworkers = ncores * _NS
  nchunk = -(-ntiles // nworkers)
  assert nb % rows == 0 and rows % 32 == 0
  mesh = plsc.VectorSubcoreMesh(
      core_axis_name="c", subcore_axis_name="s",
      num_cores=ncores, num_subcores=_NS)

  @functools.partial(
      pl.kernel,
      out_type=jax.ShapeDtypeStruct((_NSPARSE // 2, nb, _EMB), jnp.int32),
      mesh=mesh,
      scratch_types=[
          pltpu.VMEM((nchunk, chunk), jnp.float32),
          pltpu.VMEM((nchunk, chunk), jnp.int32),
          pltpu.VMEM((2, chunk, _EMB), jnp.float32),
          pltpu.VMEM((2, rows, _EMB), jnp.int32),
          pltpu.SemaphoreType.DMA,
          pltpu.SemaphoreType.DMA,
          pltpu.SemaphoreType.DMA,
          pltpu.SemaphoreType.DMA,
          pltpu.SemaphoreType.DMA,
      ],
  )
  def gather_kernel(table_hbm, xs_hbm, out_hbm, xv, idxv, bufs, cbufs,
                    gs0, gs1, cs0, cs1, xsem):
    wid = lax.axis_index("s") * ncores + lax.axis_index("c")
    gsems = (gs0, gs1)
    csems = (cs0, cs1)

    def tile_coords(j):
      t = (wid * nchunk + j) % ntiles
      fp = t // btiles
      b0 = (t % btiles) * rows
      return fp, b0

    id_copies = []
    for j in range(nchunk):
      fp, b0 = tile_coords(j)
      for s in range(2):
        id_copies.append(pltpu.async_copy(
            xs_hbm.at[2 * fp + s, pl.ds(boff + b0, rows)],
            xv.at[j, pl.ds(s * rows, rows)], xsem))
    for c in id_copies:
      c.wait()

    def conv(j, carry):
      for i in range(chunk // 16):
        v = xv[j, pl.ds(i * 16, 16)]
        idxv[j, pl.ds(i * 16, 16)] = v.astype(jnp.int32) % _VOCAB
      return carry

    lax.fori_loop(0, nchunk, conv, 0)

    def out_slice(j):
      fp, b0 = tile_coords(j)
      return out_hbm.at[fp, pl.ds(b0, rows)]

    def start_gather(j, b):
      pltpu.async_copy(table_hbm.at[idxv.at[j]], bufs.at[b], gsems[b])

    def pack_rows(b):
      def row(r, carry):
        for src_half in range(2):
          src = r + src_half * rows
          for g in range(_EMB // 32):
            lo = bufs[b, src, pl.ds(g * 32, 16)]
            hi = bufs[b, src, pl.ds(g * 32 + 16, 16)]
            ul = lax.bitcast_convert_type(lo, jnp.int32)
            uh = lax.bitcast_convert_type(hi, jnp.int32)
            wl = lax.shift_right_logical(ul + jnp.int32(0x8000), 16)
            wh = (uh + jnp.int32(0x8000)) & jnp.int32(-0x10000)
            cbufs[b, r, pl.ds(src_half * (_EMB // 2) + g * 16, 16)] = wl | wh
        return carry

      lax.fori_loop(0, rows, row, 0)

    def chunk_step(j, b, prefetch_next, wait_prev):
      pltpu.make_async_copy(
          table_hbm.at[idxv.at[j]], bufs.at[b], gsems[b]).wait()
      if prefetch_next:
        start_gather(j + 1, 1 - b)
      if wait_prev:
        pltpu.make_async_copy(cbufs.at[b], out_slice(j - 2), csems[b]).wait()
      pack_rows(b)
      pltpu.async_copy(cbufs.at[b], out_slice(j), csems[b])

    start_gather(0, 0)
    chunk_step(0, 0, True, False)
    chunk_step(1, 1, nchunk > 2, False)

    def pair(i, carry):
      for b in range(2):
        chunk_step(2 * i + b, b, True, True)
      return carry

    lax.fori_loop(1, (nchunk - 1) // 2, pair, 0)

    for j in range(2 * ((nchunk - 1) // 2), nchunk):
      if j >= 2:
        chunk_step(j, j % 2, j + 1 < nchunk, True)

    for k in (nchunk - 2, nchunk - 1):
      pltpu.make_async_copy(cbufs.at[k % 2], out_slice(k), csems[k % 2]).wait()

  return gather_kernel(table, xst)


def _mlp_body(dense_ref, emb_ref,
              bw0, bb0, bw1, bb1, bw2, bb2,
              twh, twe, tb0, tw1, tb1, tw2, tb2, tw3, tb3, tw4, tb4,
              out_ref):
  f32 = jnp.float32
  bf16 = jnp.bfloat16

  def bdot(a, w):
    return jnp.dot(a.astype(bf16), w.astype(bf16), preferred_element_type=f32)

  def fdot(a, w):
    return jnp.dot(a, w, preferred_element_type=f32)

  dense = dense_ref[...]
  h = jax.nn.relu(fdot(dense, bw0[...]) + bb0[...])
  h = jax.nn.relu(fdot(h, bw1[...]) + bb1[...]) + h
  h = jax.nn.relu(fdot(h, bw2[...]) + bb2[...]) + h
  t = fdot(h, twh[...]) + tb0[...]
  for fp in range(_NSPARSE // 2):
    w = emb_ref[fp]
    lo = lax.bitcast_convert_type(w << 16, jnp.float32)
    hi = lax.bitcast_convert_type(w & jnp.int32(-0x10000), jnp.float32)
    t += bdot(jnp.concatenate([lo, hi], axis=1), twe[fp])
  t = jax.nn.relu(t)
  t = jax.nn.relu(fdot(t, tw1[...]) + tb1[...]) + t
  t = jax.nn.relu(fdot(t, tw2[...]) + tb2[...]) + t
  t = jax.nn.relu(fdot(t, tw3[...]) + tb3[...]) + t
  out_ref[...] = fdot(t, tw4[...]) + tb4[...]


def _tc_mlp(dense, emb3, bw0, bb0, bw1, bb1, bw2, bb2,
            twh, twe, tb0, tw1, tb1, tw2, tb2, tw3, tb3, tw4, tb4):
  nb = dense.shape[0]
  bb = 512
  grid = (nb // bb,)

  def full(w):
    return pl.BlockSpec(w.shape, lambda i: (0,) * w.ndim)

  weights = (bw0, bb0, bw1, bb1, bw2, bb2,
             twh, twe, tb0, tw1, tb1, tw2, tb2, tw3, tb3, tw4, tb4)
  return pl.pallas_call(
      _mlp_body,
      grid=grid,
      in_specs=[
          pl.BlockSpec((bb, _NDENSE), lambda i: (i, 0)),
          pl.BlockSpec((_NSPARSE // 2, bb, _EMB), lambda i: (0, i, 0)),
      ] + [full(w) for w in weights],
      out_specs=pl.BlockSpec((bb, 1), lambda i: (i, 0)),
      out_shape=jax.ShapeDtypeStruct((nb, 1), jnp.float32),
  )(dense, emb3, *weights)


def kernel(x, emb_table, bot_w0, bot_b0, bot_w1, bot_b1, bot_w2, bot_b2,
           top_w0, top_b0, top_w1, top_b1, top_w2, top_b2,
           top_w3, top_b3, top_w4, top_b4):
  dense = x[:, :_NDENSE]
  xst = x[:, _NDENSE:].T
  splits = ((0, 1024), (1024, 1024), (2048, 1024), (3072, 1024))
  weights = (
      bot_w0, bot_b0.reshape(1, -1), bot_w1, bot_b1.reshape(1, -1),
      bot_w2, bot_b2.reshape(1, -1),
      top_w0[:_BOT], _twe2(top_w0[_BOT:]),
      top_b0.reshape(1, -1), top_w1, top_b1.reshape(1, -1),
      top_w2, top_b2.reshape(1, -1), top_w3, top_b3.reshape(1, -1),
      top_w4, top_b4.reshape(1, -1))
  embs = [_sc_gather(emb_table, xst, boff, nb, 64, ncores=1)
          for boff, nb in splits]
  outs = [_tc_mlp(dense[boff:boff + nb], emb, *weights)
          for (boff, nb), emb in zip(splits, embs)]
  return jnp.concatenate(outs, axis=0)


# speedup vs baseline: 1.5319x; 1.5319x over previous
import functools

import jax
import jax.numpy as jnp
import numpy as np
from jax import lax
from jax.experimental import pallas as pl
from jax.experimental.pallas import tpu as pltpu
from jax.experimental.pallas import tpu_sc as plsc

_VOCAB = 1000000
_EMB = 128
_B = 4096
_NDENSE = 13
_NSPARSE = 26
_BOT = 256

_NC, _NS = 2, 16
_NW = _NC * _NS
_TOT = _B * _NSPARSE
_PERW = _TOT // _NW
_CHUNK = 128
_NCHUNK = _PERW // _CHUNK
_BCHUNKS = _B // _CHUNK

def _twe2(tw0e):
  t = tw0e.reshape(_NSPARSE // 2, 2, 4, 2, 16, _BOT)
  return t.transpose(0, 3, 1, 2, 4, 5).reshape(_NSPARSE // 2, 2 * _EMB, _BOT)


def _sc_gather(table, xst, boff, nb, rows, ncores=_NC):
  chunk = 2 * rows
  btiles = nb // rows
  ntiles = (_NSPARSE // 2) * btiles
  nworkers = ncores * _NS
  nchunk = -(-ntiles // nworkers)
  assert nb % rows == 0 and rows % 32 == 0
  mesh = plsc.VectorSubcoreMesh(
      core_axis_name="c", subcore_axis_name="s",
      num_cores=ncores, num_subcores=_NS)

  @functools.partial(
      pl.kernel,
      out_type=jax.ShapeDtypeStruct((_NSPARSE // 2, nb, _EMB), jnp.int32),
      mesh=mesh,
      scratch_types=[
          pltpu.VMEM((nchunk, chunk), jnp.float32),
          pltpu.VMEM((nchunk, chunk), jnp.int32),
          pltpu.VMEM((2, chunk, _EMB), jnp.float32),
          pltpu.VMEM((2, rows, _EMB), jnp.int32),
          pltpu.SemaphoreType.DMA,
          pltpu.SemaphoreType.DMA,
          pltpu.SemaphoreType.DMA,
          pltpu.SemaphoreType.DMA,
          pltpu.SemaphoreType.DMA,
      ],
  )
  def gather_kernel(table_hbm, xs_hbm, out_hbm, xv, idxv, bufs, cbufs,
                    gs0, gs1, cs0, cs1, xsem):
    wid = lax.axis_index("s") * ncores + lax.axis_index("c")
    gsems = (gs0, gs1)
    csems = (cs0, cs1)

    def tile_coords(j):
      t = (wid * nchunk + j) % ntiles
      fp = t // btiles
      b0 = (t % btiles) * rows
      return fp, b0

    id_copies = []
    for j in range(nchunk):
      fp, b0 = tile_coords(j)
      for s in range(2):
        id_copies.append(pltpu.async_copy(
            xs_hbm.at[2 * fp + s, pl.ds(boff + b0, rows)],
            xv.at[j, pl.ds(s * rows, rows)], xsem))
    for c in id_copies:
      c.wait()

    def conv(j, carry):
      for i in range(chunk // 16):
        v = xv[j, pl.ds(i * 16, 16)]
        idxv[j, pl.ds(i * 16, 16)] = v.astype(jnp.int32) % _VOCAB
      return carry

    lax.fori_loop(0, nchunk, conv, 0)

    def out_slice(j):
      fp, b0 = tile_coords(j)
      return out_hbm.at[fp, pl.ds(b0, rows)]

    def start_gather(j, b):
      pltpu.async_copy(table_hbm.at[idxv.at[j]], bufs.at[b], gsems[b])

    def pack_rows(b):
      def row(r, carry):
        for src_half in range(2):
          src = r + src_half * rows
          for g in range(_EMB // 32):
            lo = bufs[b, src, pl.ds(g * 32, 16)]
            hi = bufs[b, src, pl.ds(g * 32 + 16, 16)]
            ul = lax.bitcast_convert_type(lo, jnp.int32)
            uh = lax.bitcast_convert_type(hi, jnp.int32)
            wl = lax.shift_right_logical(ul + jnp.int32(0x8000), 16)
            wh = (uh + jnp.int32(0x8000)) & jnp.int32(-0x10000)
            cbufs[b, r, pl.ds(src_half * (_EMB // 2) + g * 16, 16)] = wl | wh
        return carry

      lax.fori_loop(0, rows, row, 0)

    def chunk_step(j, b, prefetch_next, wait_prev):
      pltpu.make_async_copy(
          table_hbm.at[idxv.at[j]], bufs.at[b], gsems[b]).wait()
      if prefetch_next:
        start_gather(j + 1, 1 - b)
      if wait_prev:
        pltpu.make_async_copy(cbufs.at[b], out_slice(j - 2), csems[b]).wait()
      pack_rows(b)
      pltpu.async_copy(cbufs.at[b], out_slice(j), csems[b])

    start_gather(0, 0)
    chunk_step(0, 0, True, False)
    chunk_step(1, 1, nchunk > 2, False)

    def pair(i, carry):
      for b in range(2):
        chunk_step(2 * i + b, b, True, True)
      return carry

    lax.fori_loop(1, (nchunk - 1) // 2, pair, 0)

    for j in range(2 * ((nchunk - 1) // 2), nchunk):
      if j >= 2:
        chunk_step(j, j % 2, j + 1 < nchunk, True)

    for k in (nchunk - 2, nchunk - 1):
      pltpu.make_async_copy(cbufs.at[k % 2], out_slice(k), csems[k % 2]).wait()

  return gather_kernel(table, xst)


def _mlp_body(dense_ref, emb_ref,
              bw0, bb0, bw1, bb1, bw2, bb2,
              twh, twe, tb0, tw1, tb1, tw2, tb2, tw3, tb3, tw4, tb4,
              out_ref):
  f32 = jnp.float32
  bf16 = jnp.bfloat16

  def bdot(a, w):
    return jnp.dot(a.astype(bf16), w.astype(bf16), preferred_element_type=f32)

  def fdot(a, w):
    return jnp.dot(a, w, preferred_element_type=f32)

  dense = dense_ref[...]
  h = jax.nn.relu(fdot(dense, bw0[...]) + bb0[...])
  h = jax.nn.relu(fdot(h, bw1[...]) + bb1[...]) + h
  h = jax.nn.relu(fdot(h, bw2[...]) + bb2[...]) + h
  t = fdot(h, twh[...]) + tb0[...]
  for fp in range(_NSPARSE // 2):
    w = emb_ref[fp]
    lo = lax.bitcast_convert_type(w << 16, jnp.float32)
    hi = lax.bitcast_convert_type(w & jnp.int32(-0x10000), jnp.float32)
    t += bdot(jnp.concatenate([lo, hi], axis=1), twe[fp])
  t = jax.nn.relu(t)
  t = jax.nn.relu(fdot(t, tw1[...]) + tb1[...]) + t
  t = jax.nn.relu(fdot(t, tw2[...]) + tb2[...]) + t
  t = jax.nn.relu(fdot(t, tw3[...]) + tb3[...]) + t
  out_ref[...] = fdot(t, tw4[...]) + tb4[...]


def _tc_mlp(dense, emb3, bw0, bb0, bw1, bb1, bw2, bb2,
            twh, twe, tb0, tw1, tb1, tw2, tb2, tw3, tb3, tw4, tb4):
  nb = dense.shape[0]
  bb = 1024
  grid = (nb // bb,)

  def full(w):
    return pl.BlockSpec(w.shape, lambda i: (0,) * w.ndim)

  weights = (bw0, bb0, bw1, bb1, bw2, bb2,
             twh, twe, tb0, tw1, tb1, tw2, tb2, tw3, tb3, tw4, tb4)
  return pl.pallas_call(
      _mlp_body,
      grid=grid,
      in_specs=[
          pl.BlockSpec((bb, _NDENSE), lambda i: (i, 0)),
          pl.BlockSpec((_NSPARSE // 2, bb, _EMB), lambda i: (0, i, 0)),
      ] + [full(w) for w in weights],
      out_specs=pl.BlockSpec((bb, 1), lambda i: (i, 0)),
      out_shape=jax.ShapeDtypeStruct((nb, 1), jnp.float32),
  )(dense, emb3, *weights)


def kernel(x, emb_table, bot_w0, bot_b0, bot_w1, bot_b1, bot_w2, bot_b2,
           top_w0, top_b0, top_w1, top_b1, top_w2, top_b2,
           top_w3, top_b3, top_w4, top_b4):
  dense = x[:, :_NDENSE]
  xst = x[:, _NDENSE:].T
  splits = ((0, 2048), (2048, 2048))
  weights = (
      bot_w0, bot_b0.reshape(1, -1), bot_w1, bot_b1.reshape(1, -1),
      bot_w2, bot_b2.reshape(1, -1),
      top_w0[:_BOT], _twe2(top_w0[_BOT:]),
      top_b0.reshape(1, -1), top_w1, top_b1.reshape(1, -1),
      top_w2, top_b2.reshape(1, -1), top_w3, top_b3.reshape(1, -1),
      top_w4, top_b4.reshape(1, -1))
  embs = [_sc_gather(emb_table, xst, boff, nb, 64) for boff, nb in splits]
  outs = [_tc_mlp(dense[boff:boff + nb], emb, *weights)
          for (boff, nb), emb in zip(splits, embs)]
  return jnp.concatenate(outs, axis=0)


# speedup vs baseline: 1.5390x; 1.0046x over previous
import functools

import jax
import jax.numpy as jnp
import numpy as np
from jax import lax
from jax.experimental import pallas as pl
from jax.experimental.pallas import tpu as pltpu
from jax.experimental.pallas import tpu_sc as plsc

_VOCAB = 1000000
_EMB = 128
_B = 4096
_NDENSE = 13
_NSPARSE = 26
_BOT = 256

_NC, _NS = 2, 16
_NW = _NC * _NS
_TOT = _B * _NSPARSE
_PERW = _TOT // _NW
_CHUNK = 128
_NCHUNK = _PERW // _CHUNK
_BCHUNKS = _B // _CHUNK

def _twe2(tw0e):
  t = tw0e.reshape(_NSPARSE // 2, 2, 4, 2, 16, _BOT)
  return t.transpose(0, 3, 1, 2, 4, 5).reshape(_NSPARSE // 2, 2 * _EMB, _BOT)


def _sc_gather(table, xst, boff, nb, rows, ncores=_NC):
  chunk = 2 * rows
  btiles = nb // rows
  ntiles = (_NSPARSE // 2) * btiles
  nworkers = ncores * _NS
  nchunk = -(-ntiles // nworkers)
  assert nb % rows == 0 and rows % 32 == 0
  mesh = plsc.VectorSubcoreMesh(
      core_axis_name="c", subcore_axis_name="s",
      num_cores=ncores, num_subcores=_NS)

  @functools.partial(
      pl.kernel,
      out_type=jax.ShapeDtypeStruct((_NSPARSE // 2, nb, _EMB), jnp.int32),
      mesh=mesh,
      scratch_types=[
          pltpu.VMEM((nchunk, chunk), jnp.float32),
          pltpu.VMEM((nchunk, chunk), jnp.int32),
          pltpu.VMEM((2, chunk, _EMB), jnp.float32),
          pltpu.VMEM((2, rows, _EMB), jnp.int32),
          pltpu.SemaphoreType.DMA,
          pltpu.SemaphoreType.DMA,
          pltpu.SemaphoreType.DMA,
          pltpu.SemaphoreType.DMA,
          pltpu.SemaphoreType.DMA,
      ],
  )
  def gather_kernel(table_hbm, xs_hbm, out_hbm, xv, idxv, bufs, cbufs,
                    gs0, gs1, cs0, cs1, xsem):
    wid = lax.axis_index("s") * ncores + lax.axis_index("c")
    gsems = (gs0, gs1)
    csems = (cs0, cs1)

    def tile_coords(j):
      t = (wid * nchunk + j) % ntiles
      fp = t // btiles
      b0 = (t % btiles) * rows
      return fp, b0

    id_copies = []
    for j in range(nchunk):
      fp, b0 = tile_coords(j)
      for s in range(2):
        id_copies.append(pltpu.async_copy(
            xs_hbm.at[2 * fp + s, pl.ds(boff + b0, rows)],
            xv.at[j, pl.ds(s * rows, rows)], xsem))
    for c in id_copies:
      c.wait()

    def conv(j, carry):
      for i in range(chunk // 16):
        v = xv[j, pl.ds(i * 16, 16)]
        idxv[j, pl.ds(i * 16, 16)] = v.astype(jnp.int32) % _VOCAB
      return carry

    lax.fori_loop(0, nchunk, conv, 0)

    def out_slice(j):
      fp, b0 = tile_coords(j)
      return out_hbm.at[fp, pl.ds(b0, rows)]

    def start_gather(j, b):
      pltpu.async_copy(table_hbm.at[idxv.at[j]], bufs.at[b], gsems[b])

    def pack_rows(b):
      def row(r, carry):
        for src_half in range(2):
          src = r + src_half * rows
          for g in range(_EMB // 32):
            lo = bufs[b, src, pl.ds(g * 32, 16)]
            hi = bufs[b, src, pl.ds(g * 32 + 16, 16)]
            ul = lax.bitcast_convert_type(lo, jnp.int32)
            uh = lax.bitcast_convert_type(hi, jnp.int32)
            wl = lax.shift_right_logical(ul + jnp.int32(0x8000), 16)
            wh = (uh + jnp.int32(0x8000)) & jnp.int32(-0x10000)
            cbufs[b, r, pl.ds(src_half * (_EMB // 2) + g * 16, 16)] = wl | wh
        return carry

      lax.fori_loop(0, rows, row, 0)

    def chunk_step(j, b, prefetch_next, wait_prev):
      pltpu.make_async_copy(
          table_hbm.at[idxv.at[j]], bufs.at[b], gsems[b]).wait()
      if prefetch_next:
        start_gather(j + 1, 1 - b)
      if wait_prev:
        pltpu.make_async_copy(cbufs.at[b], out_slice(j - 2), csems[b]).wait()
      pack_rows(b)
      pltpu.async_copy(cbufs.at[b], out_slice(j), csems[b])

    start_gather(0, 0)
    chunk_step(0, 0, True, False)
    chunk_step(1, 1, nchunk > 2, False)

    def pair(i, carry):
      for b in range(2):
        chunk_step(2 * i + b, b, True, True)
      return carry

    lax.fori_loop(1, (nchunk - 1) // 2, pair, 0)

    for j in range(2 * ((nchunk - 1) // 2), nchunk):
      if j >= 2:
        chunk_step(j, j % 2, j + 1 < nchunk, True)

    for k in (nchunk - 2, nchunk - 1):
      pltpu.make_async_copy(cbufs.at[k % 2], out_slice(k), csems[k % 2]).wait()

  return gather_kernel(table, xst)


def _mlp_body(dense_ref, emb_ref,
              bw0, bb0, bw1, bb1, bw2, bb2,
              twh, twe, tb0, tw1, tb1, tw2, tb2, tw3, tb3, tw4, tb4,
              out_ref):
  f32 = jnp.float32
  bf16 = jnp.bfloat16

  def bdot(a, w):
    return jnp.dot(a.astype(bf16), w.astype(bf16), preferred_element_type=f32)

  def fdot(a, w):
    return jnp.dot(a, w, preferred_element_type=f32)

  dense = dense_ref[...]
  h = jax.nn.relu(fdot(dense, bw0[...]) + bb0[...])
  h = jax.nn.relu(fdot(h, bw1[...]) + bb1[...]) + h
  h = jax.nn.relu(fdot(h, bw2[...]) + bb2[...]) + h
  t = fdot(h, twh[...]) + tb0[...]
  for fp in range(_NSPARSE // 2):
    w = emb_ref[fp]
    lo = lax.bitcast_convert_type(w << 16, jnp.float32)
    hi = lax.bitcast_convert_type(w & jnp.int32(-0x10000), jnp.float32)
    t += bdot(jnp.concatenate([lo, hi], axis=1), twe[fp])
  t = jax.nn.relu(t)
  t = jax.nn.relu(fdot(t, tw1[...]) + tb1[...]) + t
  t = jax.nn.relu(fdot(t, tw2[...]) + tb2[...]) + t
  t = jax.nn.relu(fdot(t, tw3[...]) + tb3[...]) + t
  out_ref[...] = fdot(t, tw4[...]) + tb4[...]


def _tc_mlp(dense, emb3, bw0, bb0, bw1, bb1, bw2, bb2,
            twh, twe, tb0, tw1, tb1, tw2, tb2, tw3, tb3, tw4, tb4):
  nb = dense.shape[0]
  bb = 512
  grid = (nb // bb,)

  def full(w):
    return pl.BlockSpec(w.shape, lambda i: (0,) * w.ndim)

  weights = (bw0, bb0, bw1, bb1, bw2, bb2,
             twh, twe, tb0, tw1, tb1, tw2, tb2, tw3, tb3, tw4, tb4)
  return pl.pallas_call(
      _mlp_body,
      grid=grid,
      in_specs=[
          pl.BlockSpec((bb, _NDENSE), lambda i: (i, 0)),
          pl.BlockSpec((_NSPARSE // 2, bb, _EMB), lambda i: (0, i, 0)),
      ] + [full(w) for w in weights],
      out_specs=pl.BlockSpec((bb, 1), lambda i: (i, 0)),
      out_shape=jax.ShapeDtypeStruct((nb, 1), jnp.float32),
  )(dense, emb3, *weights)


def kernel(x, emb_table, bot_w0, bot_b0, bot_w1, bot_b1, bot_w2, bot_b2,
           top_w0, top_b0, top_w1, top_b1, top_w2, top_b2,
           top_w3, top_b3, top_w4, top_b4):
  dense = x[:, :_NDENSE]
  xst = x[:, _NDENSE:].T
  splits = ((0, 2048), (2048, 2048))
  weights = (
      bot_w0, bot_b0.reshape(1, -1), bot_w1, bot_b1.reshape(1, -1),
      bot_w2, bot_b2.reshape(1, -1),
      top_w0[:_BOT], _twe2(top_w0[_BOT:]),
      top_b0.reshape(1, -1), top_w1, top_b1.reshape(1, -1),
      top_w2, top_b2.reshape(1, -1), top_w3, top_b3.reshape(1, -1),
      top_w4, top_b4.reshape(1, -1))
  embs = [_sc_gather(emb_table, xst, boff, nb, 64) for boff, nb in splits]
  outs = [_tc_mlp(dense[boff:boff + nb], emb, *weights)
          for (boff, nb), emb in zip(splits, embs)]
  return jnp.concatenate(outs, axis=0)
